# baseline (speedup 1.0000x reference)
import jax
import jax.numpy as jnp
from jax import lax
from jax.experimental import pallas as pl
from jax.experimental.pallas import tpu as pltpu
from jax.experimental.pallas import tpu_sc as plsc

NUM_EMB = 1000000
DIM = 64
BATCH = 16384
N_IDX = 327680

NC = 2
NS = 16
NW = NC * NS
BAGS_W = BATCH // NW
BLK = BAGS_W + 8
DUMP = BAGS_W
SEG_P = 16384
SEG_ROWS = SEG_P // 128
NBV = SEG_P // 16 // 16
OFFV = 528
OFF_PAD = 31 * BAGS_W + OFFV
IDX_PAD = N_IDX + SEG_P


def _sc_body(weight, indices, offsets, out, idx_v, seg_v, off_v, bv_v, pv_v,
             rows_v, acc, sem0, sem1):
    c = lax.axis_index("c")
    s = lax.axis_index("s")
    wid = c * NS + s
    blk0 = s * BLK

    iota16 = lax.iota(jnp.int32, 16)
    zi16 = jnp.zeros((16,), jnp.int32)
    zf16 = jnp.zeros((16,), jnp.float32)
    ones16 = jnp.ones((16,), jnp.int32)

    def zrow(t, _):
        rows_v[0, t >> 2, pl.ds((t & 3) * 16, 16)] = zf16
        return 0

    lax.fori_loop(0, 128 * DIM // 16, zrow, 0)
    for k in range(4):
        pltpu.sync_copy(rows_v.at[0], acc.at[pl.ds(blk0 + k * 128, 128)])
    pltpu.sync_copy(rows_v.at[0, pl.ds(0, 8)], acc.at[pl.ds(blk0 + 512, 8)])

    pltpu.sync_copy(offsets.at[pl.ds(wid * BAGS_W, OFFV)], off_v)
    first = off_v[pl.ds(0, 16)][0]
    last = off_v[pl.ds(512, 16)][0]
    pstart = jnp.where(wid == 0, 0, first)
    pend = jnp.where(wid == NW - 1, N_IDX, last)
    a0 = (pstart >> 7) << 7
    nsc = jnp.maximum(0, (pend - a0 + SEG_P - 1) >> 14)

    def superchunk(sc_i, _):
        a0_sc = pl.multiple_of(a0 + sc_i * SEG_P, 128)

        def zseg(t, _):
            seg_v[t >> 3, pl.ds((t & 7) * 16, 16)] = zi16
            return 0

        lax.fori_loop(0, SEG_P // 16, zseg, 0)

        def zb(u, _):
            bv_v[pl.ds(u * 16, 16)] = zi16
            return 0

        lax.fori_loop(0, NBV, zb, 0)

        pltpu.sync_copy(indices.at[pl.ds(a0_sc, SEG_P)], idx_v)

        def remap(t, _):
            v = idx_v[pl.ds(t * 16, 16)]
            r = v & 4095
            h = jnp.where(r >= 2048, ones16, zi16)
            idx_v[pl.ds(t * 16, 16)] = (v - r) + ((r & 2047) << 1) + h
            return 0

        lax.fori_loop(0, SEG_P // 16, remap, 0)

        def do_boundary(k, v_k, base):
            v_k = jnp.where(
                jnp.logical_and(k == BAGS_W, wid == NW - 1), N_IDX, v_k)
            rel = v_k - a0_sc
            below = rel < 0
            inr = jnp.logical_and(rel >= 0, rel < SEG_P)
            relc = jnp.clip(rel, 0, SEG_P - 1)
            row = relc >> 7
            cg = (relc >> 4) & 7
            lane = relc & 15
            inr_i = jnp.where(inr, 1, 0)
            cur = seg_v[row, pl.ds(cg * 16, 16)]
            ramp = jnp.where(iota16 >= lane, ones16, zi16) * inr_i
            seg_v[row, pl.ds(cg * 16, 16)] = cur + ramp
            tv = relc >> 4
            boff = (tv >> 4) * 16
            bl = tv & 15
            bcur = bv_v[pl.ds(boff, 16)]
            binc = jnp.where(iota16 == bl, ones16, zi16) * inr_i
            bv_v[pl.ds(boff, 16)] = bcur + binc
            return base + jnp.where(below, 1, 0)

        def bgroup(kk, base):
            va = off_v[pl.ds(kk * 16, 16)]
            for u in range(16):
                k = kk * 16 + u
                is_valid = jnp.logical_and(k >= 1, k <= BAGS_W)
                v_k = jnp.where(is_valid, va[u], jnp.int32(2 * N_IDX))
                base = do_boundary(k, v_k, base)
            return base

        base = lax.fori_loop(0, (BAGS_W + 16) // 16, bgroup, jnp.int32(0))

        def prefix(u, carry):
            bv = bv_v[pl.ds(u * 16, 16)]
            incl = bv
            for sh in (1, 2, 4, 8):
                shifted = jnp.take(incl, jnp.clip(iota16 - sh, 0, 15))
                incl = incl + jnp.where(iota16 >= sh, shifted, zi16)
            pv_v[pl.ds(u * 16, 16)] = (incl - bv) + carry
            return carry + incl[15]

        lax.fori_loop(0, NBV, prefix, base)

        def finalize(g, _):
            pv = pv_v[pl.ds(g * 16, 16)]
            for u in range(16):
                t = g * 16 + u
                row = t >> 3
                cg = t & 7
                val = seg_v[row, pl.ds(cg * 16, 16)] + pv[u]
                p16 = (a0_sc + t * 16) + iota16
                val = jnp.where(p16 < pstart,
                                jnp.full((16,), DUMP, jnp.int32), val)
                seg_v[row, pl.ds(cg * 16, 16)] = val + blk0
            return 0

        lax.fori_loop(0, NBV, finalize, 0)

        nch = jnp.clip((pend - a0_sc + 127) >> 7, 0, SEG_ROWS)

        def chunk(j, _):
            pltpu.sync_copy(weight.at[idx_v.at[pl.ds(j * 128, 128)]],
                            rows_v.at[0])
            pltpu.sync_copy(rows_v.at[0], acc.at[seg_v.at[j]], add=True)
            return 0

        lax.fori_loop(0, nch, chunk, 0)
        return 0

    lax.fori_loop(0, nsc, superchunk, 0)

    pltpu.sync_copy(acc.at[pl.ds(blk0, BAGS_W)],
                    out.at[pl.ds(wid * BAGS_W, BAGS_W)])


TBLK = 2048
GROUPS = (NUM_EMB + 2 * TBLK - 1) // (2 * TBLK)
V_ROWS = GROUPS * 2 * TBLK


NBLK = NUM_EMB // TBLK


def _tc_transpose_body(a1_ref, a2_ref, alast_ref, o_ref):
    g = pl.program_id(0)
    eye = jnp.eye(DIM, dtype=jnp.float32)
    dn = (((0,), (0,)), ((), ()))
    a1 = jnp.where(g == GROUPS - 1, alast_ref[...], a1_ref[...])
    o_ref[:, 0:DIM] = lax.dot_general(
        a1, eye, dn, precision=lax.Precision.HIGHEST,
        preferred_element_type=jnp.float32)
    a2t = lax.dot_general(
        a2_ref[...], eye, dn, precision=lax.Precision.HIGHEST,
        preferred_element_type=jnp.float32)
    o_ref[:, DIM:2 * DIM] = jnp.where(g == GROUPS - 1,
                                      jnp.zeros_like(a2t), a2t)


def _relayout(weight):
    wt = weight.T
    tail = jax.lax.slice(wt, (0, NBLK * TBLK), (DIM, NUM_EMB))
    wlast = jnp.concatenate(
        [tail, jnp.zeros((DIM, TBLK - (NUM_EMB - NBLK * TBLK)), jnp.float32)],
        axis=1)
    paired = pl.pallas_call(
        _tc_transpose_body,
        out_shape=jax.ShapeDtypeStruct((GROUPS * TBLK, 2 * DIM), jnp.float32),
        grid=(GROUPS,),
        in_specs=[
            pl.BlockSpec((DIM, TBLK),
                         lambda i: (0, jnp.minimum(2 * i, NBLK - 1))),
            pl.BlockSpec((DIM, TBLK),
                         lambda i: (0, jnp.minimum(2 * i + 1, NBLK - 1))),
            pl.BlockSpec((DIM, TBLK), lambda i: (0, 0)),
        ],
        out_specs=pl.BlockSpec((TBLK, 2 * DIM), lambda i: (i, 0)),
    )(wt, wt, wlast)
    return paired.reshape(V_ROWS, DIM)


@jax.jit
def kernel(weight, indices, offsets):
    indices = indices.astype(jnp.int32)
    offsets = offsets.astype(jnp.int32)
    weight_rm = _relayout(weight)
    indices_p = jnp.concatenate(
        [indices, jnp.zeros((IDX_PAD - N_IDX,), jnp.int32)])
    offsets_p = jnp.concatenate(
        [offsets, jnp.full((OFF_PAD - (BATCH + 1),), N_IDX, jnp.int32)])

    mesh = plsc.VectorSubcoreMesh(core_axis_name="c", subcore_axis_name="s")
    out = pl.kernel(
        _sc_body,
        out_type=jax.ShapeDtypeStruct((BATCH, DIM), jnp.float32),
        mesh=mesh,
        compiler_params=pltpu.CompilerParams(use_tc_tiling_on_sc=False),
        scratch_types=[
            pltpu.VMEM((SEG_P,), jnp.int32),
            pltpu.VMEM((SEG_ROWS, 128), jnp.int32),
            pltpu.VMEM((OFFV,), jnp.int32),
            pltpu.VMEM((SEG_P // 16,), jnp.int32),
            pltpu.VMEM((SEG_P // 16,), jnp.int32),
            pltpu.VMEM((2, 128, DIM), jnp.float32),
            pltpu.VMEM_SHARED((NS * BLK, DIM), jnp.float32),
            pltpu.SemaphoreType.DMA,
            pltpu.SemaphoreType.DMA,
        ],
    )(weight_rm, indices_p, offsets_p)
    return out


# speedup vs baseline: 1.6474x; 1.6474x over previous
import jax
import jax.numpy as jnp
from jax import lax
from jax.experimental import pallas as pl
from jax.experimental.pallas import tpu as pltpu
from jax.experimental.pallas import tpu_sc as plsc

NUM_EMB = 1000000
DIM = 64
BATCH = 16384
N_IDX = 327680

NC = 2
NS = 16
NW = NC * NS
BAGS_W = BATCH // NW
BLK = BAGS_W + 8
DUMP = BAGS_W
SEG_P = 16384
SEG_ROWS = SEG_P // 128
NBV = SEG_P // 16 // 16
OFFV = 528
OFF_PAD = 31 * BAGS_W + OFFV
IDX_PAD = N_IDX + SEG_P


def _sc_body(weight, indices, offsets, out, idx_v, seg_v, off_v, bv_v, pv_v,
             rows_v, acc, sem0, sem1):
    c = lax.axis_index("c")
    s = lax.axis_index("s")
    wid = c * NS + s
    blk0 = s * BLK

    iota16 = lax.iota(jnp.int32, 16)
    zi16 = jnp.zeros((16,), jnp.int32)
    zf16 = jnp.zeros((16,), jnp.float32)
    ones16 = jnp.ones((16,), jnp.int32)

    def zrow(t, _):
        rows_v[0, t >> 2, pl.ds((t & 3) * 16, 16)] = zf16
        return 0

    lax.fori_loop(0, 128 * DIM // 16, zrow, 0)
    for k in range(4):
        pltpu.sync_copy(rows_v.at[0], acc.at[pl.ds(blk0 + k * 128, 128)])
    pltpu.sync_copy(rows_v.at[0, pl.ds(0, 8)], acc.at[pl.ds(blk0 + 512, 8)])

    pltpu.sync_copy(offsets.at[pl.ds(wid * BAGS_W, OFFV)], off_v)
    first = off_v[pl.ds(0, 16)][0]
    last = off_v[pl.ds(512, 16)][0]
    pstart = jnp.where(wid == 0, 0, first)
    pend = jnp.where(wid == NW - 1, N_IDX, last)
    a0 = (pstart >> 7) << 7
    nsc = jnp.maximum(0, (pend - a0 + SEG_P - 1) >> 14)

    def superchunk(sc_i, _):
        a0_sc = pl.multiple_of(a0 + sc_i * SEG_P, 128)

        def zseg(t, _):
            seg_v[t >> 3, pl.ds((t & 7) * 16, 16)] = zi16
            return 0

        lax.fori_loop(0, SEG_P // 16, zseg, 0)

        def zb(u, _):
            bv_v[pl.ds(u * 16, 16)] = zi16
            return 0

        lax.fori_loop(0, NBV, zb, 0)

        pltpu.sync_copy(indices.at[pl.ds(a0_sc, SEG_P)], idx_v)

        def remap(t, _):
            v = idx_v[pl.ds(t * 16, 16)]
            r = v & 4095
            h = jnp.where(r >= 2048, ones16, zi16)
            idx_v[pl.ds(t * 16, 16)] = (v - r) + ((r & 2047) << 1) + h
            return 0

        lax.fori_loop(0, SEG_P // 16, remap, 0)

        def do_boundary(k, v_k, base):
            v_k = jnp.where(
                jnp.logical_and(k == BAGS_W, wid == NW - 1), N_IDX, v_k)
            rel = v_k - a0_sc
            below = rel < 0
            inr = jnp.logical_and(rel >= 0, rel < SEG_P)
            relc = jnp.clip(rel, 0, SEG_P - 1)
            row = relc >> 7
            cg = (relc >> 4) & 7
            lane = relc & 15
            inr_i = jnp.where(inr, 1, 0)
            cur = seg_v[row, pl.ds(cg * 16, 16)]
            ramp = jnp.where(iota16 >= lane, ones16, zi16) * inr_i
            seg_v[row, pl.ds(cg * 16, 16)] = cur + ramp
            tv = relc >> 4
            boff = (tv >> 4) * 16
            bl = tv & 15
            bcur = bv_v[pl.ds(boff, 16)]
            binc = jnp.where(iota16 == bl, ones16, zi16) * inr_i
            bv_v[pl.ds(boff, 16)] = bcur + binc
            return base + jnp.where(below, 1, 0)

        def bgroup(kk, base):
            va = off_v[pl.ds(kk * 16, 16)]
            for u in range(16):
                k = kk * 16 + u
                is_valid = jnp.logical_and(k >= 1, k <= BAGS_W)
                v_k = jnp.where(is_valid, va[u], jnp.int32(2 * N_IDX))
                base = do_boundary(k, v_k, base)
            return base

        base = lax.fori_loop(0, (BAGS_W + 16) // 16, bgroup, jnp.int32(0))

        def prefix(u, carry):
            bv = bv_v[pl.ds(u * 16, 16)]
            incl = bv
            for sh in (1, 2, 4, 8):
                shifted = jnp.take(incl, jnp.clip(iota16 - sh, 0, 15))
                incl = incl + jnp.where(iota16 >= sh, shifted, zi16)
            pv_v[pl.ds(u * 16, 16)] = (incl - bv) + carry
            return carry + incl[15]

        lax.fori_loop(0, NBV, prefix, base)

        def finalize(g, _):
            pv = pv_v[pl.ds(g * 16, 16)]
            for u in range(16):
                t = g * 16 + u
                row = t >> 3
                cg = t & 7
                val = seg_v[row, pl.ds(cg * 16, 16)] + pv[u]
                p16 = (a0_sc + t * 16) + iota16
                val = jnp.where(p16 < pstart,
                                jnp.full((16,), DUMP, jnp.int32), val)
                seg_v[row, pl.ds(cg * 16, 16)] = val + blk0
            return 0

        lax.fori_loop(0, NBV, finalize, 0)

        nch = jnp.clip((pend - a0_sc + 127) >> 7, 0, SEG_ROWS)

        def chunk(j, _):
            pltpu.sync_copy(weight.at[idx_v.at[pl.ds(j * 128, 128)]],
                            rows_v.at[0])
            pltpu.sync_copy(rows_v.at[0], acc.at[seg_v.at[j]], add=True)
            return 0

        lax.fori_loop(0, nch, chunk, 0)
        return 0

    lax.fori_loop(0, nsc, superchunk, 0)

    pltpu.sync_copy(acc.at[pl.ds(blk0, BAGS_W)],
                    out.at[pl.ds(wid * BAGS_W, BAGS_W)])


TBLK = 2048
GROUPS = (NUM_EMB + 2 * TBLK - 1) // (2 * TBLK)


STEP_COLS = 4 * TBLK
NFULL = NUM_EMB // STEP_COLS
NSTEP = GROUPS // 2 + 1
V_ROWS = NSTEP * STEP_COLS


def _tc_transpose_body(a_ref, last_ref, o_ref):
    i = pl.program_id(0)
    src = jnp.where(i == NSTEP - 1, last_ref[...], a_ref[...])
    for j in range(2):
        o_ref[pl.ds(j * TBLK, TBLK), 0:DIM] = (
            src[:, j * 2 * TBLK:j * 2 * TBLK + TBLK].T)
        o_ref[pl.ds(j * TBLK, TBLK), DIM:2 * DIM] = (
            src[:, j * 2 * TBLK + TBLK:(j + 1) * 2 * TBLK].T)


def _relayout(weight):
    wt = weight.T
    tail = jax.lax.slice(wt, (0, NFULL * STEP_COLS), (DIM, NUM_EMB))
    wlast = jnp.concatenate(
        [tail,
         jnp.zeros((DIM, STEP_COLS - (NUM_EMB - NFULL * STEP_COLS)),
                   jnp.float32)], axis=1)
    paired = pl.pallas_call(
        _tc_transpose_body,
        out_shape=jax.ShapeDtypeStruct((NSTEP * 2 * TBLK, 2 * DIM),
                                       jnp.float32),
        grid=(NSTEP,),
        in_specs=[
            pl.BlockSpec((DIM, STEP_COLS),
                         lambda i: (0, jnp.minimum(i, NFULL - 1))),
            pl.BlockSpec((DIM, STEP_COLS), lambda i: (0, 0)),
        ],
        out_specs=pl.BlockSpec((2 * TBLK, 2 * DIM), lambda i: (i, 0)),
    )(wt, wlast)
    return paired.reshape(V_ROWS, DIM)


@jax.jit
def kernel(weight, indices, offsets):
    indices = indices.astype(jnp.int32)
    offsets = offsets.astype(jnp.int32)
    weight_rm = _relayout(weight)
    indices_p = jnp.concatenate(
        [indices, jnp.zeros((IDX_PAD - N_IDX,), jnp.int32)])
    offsets_p = jnp.concatenate(
        [offsets, jnp.full((OFF_PAD - (BATCH + 1),), N_IDX, jnp.int32)])

    mesh = plsc.VectorSubcoreMesh(core_axis_name="c", subcore_axis_name="s")
    out = pl.kernel(
        _sc_body,
        out_type=jax.ShapeDtypeStruct((BATCH, DIM), jnp.float32),
        mesh=mesh,
        compiler_params=pltpu.CompilerParams(use_tc_tiling_on_sc=False),
        scratch_types=[
            pltpu.VMEM((SEG_P,), jnp.int32),
            pltpu.VMEM((SEG_ROWS, 128), jnp.int32),
            pltpu.VMEM((OFFV,), jnp.int32),
            pltpu.VMEM((SEG_P // 16,), jnp.int32),
            pltpu.VMEM((SEG_P // 16,), jnp.int32),
            pltpu.VMEM((2, 128, DIM), jnp.float32),
            pltpu.VMEM_SHARED((NS * BLK, DIM), jnp.float32),
            pltpu.SemaphoreType.DMA,
            pltpu.SemaphoreType.DMA,
        ],
    )(weight_rm, indices_p, offsets_p)
    return out


# speedup vs baseline: 1.8762x; 1.1389x over previous
import jax
import jax.numpy as jnp
from jax import lax
from jax.experimental import pallas as pl
from jax.experimental.pallas import tpu as pltpu
from jax.experimental.pallas import tpu_sc as plsc

NUM_EMB = 1000000
DIM = 64
BATCH = 16384
N_IDX = 327680

NC = 2
NS = 16
NW = NC * NS
BAGS_W = BATCH // NW
BLK = BAGS_W + 8
DUMP = BAGS_W
SEG_P = 16384
SEG_ROWS = SEG_P // 128
NBV = SEG_P // 16 // 16
OFFV = 528
OFF_PAD = 31 * BAGS_W + OFFV
IDX_PAD = N_IDX + SEG_P


def _sc_body(weight, indices, offsets, out, idx_v, seg_v, off_v, bv_v, pv_v,
             rows_v, acc, sem0, sem1):
    c = lax.axis_index("c")
    s = lax.axis_index("s")
    wid = c * NS + s
    blk0 = s * BLK

    iota16 = lax.iota(jnp.int32, 16)
    zi16 = jnp.zeros((16,), jnp.int32)
    zf16 = jnp.zeros((16,), jnp.float32)
    ones16 = jnp.ones((16,), jnp.int32)

    def zrow(t, _):
        rows_v[0, t >> 2, pl.ds((t & 3) * 16, 16)] = zf16
        return 0

    lax.fori_loop(0, 128 * DIM // 16, zrow, 0)
    for k in range(4):
        pltpu.sync_copy(rows_v.at[0], acc.at[pl.ds(blk0 + k * 128, 128)])
    pltpu.sync_copy(rows_v.at[0, pl.ds(0, 8)], acc.at[pl.ds(blk0 + 512, 8)])

    pltpu.sync_copy(offsets.at[pl.ds(wid * BAGS_W, OFFV)], off_v)
    first = off_v[pl.ds(0, 16)][0]
    last = off_v[pl.ds(512, 16)][0]
    pstart = jnp.where(wid == 0, 0, first)
    pend = jnp.where(wid == NW - 1, N_IDX, last)
    a0 = (pstart >> 7) << 7
    nsc = jnp.maximum(0, (pend - a0 + SEG_P - 1) >> 14)

    def superchunk(sc_i, _):
        a0_sc = pl.multiple_of(a0 + sc_i * SEG_P, 128)

        def zseg(t, _):
            seg_v[t >> 3, pl.ds((t & 7) * 16, 16)] = zi16
            return 0

        lax.fori_loop(0, SEG_P // 16, zseg, 0)

        def zb(u, _):
            bv_v[pl.ds(u * 16, 16)] = zi16
            return 0

        lax.fori_loop(0, NBV, zb, 0)

        pltpu.sync_copy(indices.at[pl.ds(a0_sc, SEG_P)], idx_v)

        def remap(t, _):
            v = idx_v[pl.ds(t * 16, 16)]
            r = v & 4095
            h = jnp.where(r >= 2048, ones16, zi16)
            idx_v[pl.ds(t * 16, 16)] = (v - r) + ((r & 2047) << 1) + h
            return 0

        lax.fori_loop(0, SEG_P // 16, remap, 0)

        def do_boundary(k, v_k, base):
            v_k = jnp.where(
                jnp.logical_and(k == BAGS_W, wid == NW - 1), N_IDX, v_k)
            rel = v_k - a0_sc
            below = rel < 0
            inr = jnp.logical_and(rel >= 0, rel < SEG_P)
            relc = jnp.clip(rel, 0, SEG_P - 1)
            row = relc >> 7
            cg = (relc >> 4) & 7
            lane = relc & 15
            inr_i = jnp.where(inr, 1, 0)
            cur = seg_v[row, pl.ds(cg * 16, 16)]
            ramp = jnp.where(iota16 >= lane, ones16, zi16) * inr_i
            seg_v[row, pl.ds(cg * 16, 16)] = cur + ramp
            tv = relc >> 4
            boff = (tv >> 4) * 16
            bl = tv & 15
            bcur = bv_v[pl.ds(boff, 16)]
            binc = jnp.where(iota16 == bl, ones16, zi16) * inr_i
            bv_v[pl.ds(boff, 16)] = bcur + binc
            return base + jnp.where(below, 1, 0)

        def bgroup(kk, base):
            va = off_v[pl.ds(kk * 16, 16)]
            for u in range(16):
                k = kk * 16 + u
                is_valid = jnp.logical_and(k >= 1, k <= BAGS_W)
                v_k = jnp.where(is_valid, va[u], jnp.int32(2 * N_IDX))
                base = do_boundary(k, v_k, base)
            return base

        base = lax.fori_loop(0, (BAGS_W + 16) // 16, bgroup, jnp.int32(0))

        def prefix(u, carry):
            bv = bv_v[pl.ds(u * 16, 16)]
            incl = bv
            for sh in (1, 2, 4, 8):
                shifted = jnp.take(incl, jnp.clip(iota16 - sh, 0, 15))
                incl = incl + jnp.where(iota16 >= sh, shifted, zi16)
            pv_v[pl.ds(u * 16, 16)] = (incl - bv) + carry
            return carry + incl[15]

        lax.fori_loop(0, NBV, prefix, base)

        def finalize(g, _):
            pv = pv_v[pl.ds(g * 16, 16)]
            for u in range(16):
                t = g * 16 + u
                row = t >> 3
                cg = t & 7
                val = seg_v[row, pl.ds(cg * 16, 16)] + pv[u]
                p16 = (a0_sc + t * 16) + iota16
                val = jnp.where(p16 < pstart,
                                jnp.full((16,), DUMP, jnp.int32), val)
                seg_v[row, pl.ds(cg * 16, 16)] = val + blk0
            return 0

        lax.fori_loop(0, NBV, finalize, 0)

        nch = jnp.clip((pend - a0_sc + 127) >> 7, 0, SEG_ROWS)
        nch2 = nch >> 1

        def g_issue(j, b_ref, sem):
            pltpu.async_copy(
                weight.at[idx_v.at[pl.ds(j * 128, 128)]], b_ref, sem)

        def g_wait(b_ref, sem):
            pltpu.make_async_copy(
                weight.at[idx_v.at[pl.ds(0, 128)]], b_ref, sem).wait()

        g_issue(0, rows_v.at[0], sem0)

        def pair(m, _):
            j0 = m * 2
            g_issue(j0 + 1, rows_v.at[1], sem1)
            g_wait(rows_v.at[0], sem0)
            pltpu.sync_copy(rows_v.at[0], acc.at[seg_v.at[j0]], add=True)
            g_issue(jnp.minimum(j0 + 2, nch - 1), rows_v.at[0], sem0)
            g_wait(rows_v.at[1], sem1)
            pltpu.sync_copy(rows_v.at[1], acc.at[seg_v.at[j0 + 1]], add=True)
            return 0

        lax.fori_loop(0, nch2, pair, 0)

        def odd(_t, _c):
            g_wait(rows_v.at[0], sem0)
            pltpu.sync_copy(rows_v.at[0], acc.at[seg_v.at[nch - 1]], add=True)
            return 0

        lax.fori_loop(0, nch & 1, odd, 0)

        def drain(_t, _c):
            g_wait(rows_v.at[0], sem0)
            return 0

        lax.fori_loop(0, ((nch & 1) ^ 1) * jnp.minimum(nch2, 1), drain, 0)
        return 0

    lax.fori_loop(0, nsc, superchunk, 0)

    pltpu.sync_copy(acc.at[pl.ds(blk0, BAGS_W)],
                    out.at[pl.ds(wid * BAGS_W, BAGS_W)])


TBLK = 2048
GROUPS = (NUM_EMB + 2 * TBLK - 1) // (2 * TBLK)


STEP_COLS = 4 * TBLK
NFULL = NUM_EMB // STEP_COLS
NSTEP = GROUPS // 2 + 1
V_ROWS = NSTEP * STEP_COLS


def _tc_transpose_body(a_ref, last_ref, o_ref):
    i = pl.program_id(0)
    src = jnp.where(i == NSTEP - 1, last_ref[...], a_ref[...])
    for j in range(2):
        o_ref[pl.ds(j * TBLK, TBLK), 0:DIM] = (
            src[:, j * 2 * TBLK:j * 2 * TBLK + TBLK].T)
        o_ref[pl.ds(j * TBLK, TBLK), DIM:2 * DIM] = (
            src[:, j * 2 * TBLK + TBLK:(j + 1) * 2 * TBLK].T)


def _relayout(weight):
    wt = weight.T
    tail = jax.lax.slice(wt, (0, NFULL * STEP_COLS), (DIM, NUM_EMB))
    wlast = jnp.concatenate(
        [tail,
         jnp.zeros((DIM, STEP_COLS - (NUM_EMB - NFULL * STEP_COLS)),
                   jnp.float32)], axis=1)
    paired = pl.pallas_call(
        _tc_transpose_body,
        out_shape=jax.ShapeDtypeStruct((NSTEP * 2 * TBLK, 2 * DIM),
                                       jnp.float32),
        grid=(NSTEP,),
        in_specs=[
            pl.BlockSpec((DIM, STEP_COLS),
                         lambda i: (0, jnp.minimum(i, NFULL - 1))),
            pl.BlockSpec((DIM, STEP_COLS), lambda i: (0, 0)),
        ],
        out_specs=pl.BlockSpec((2 * TBLK, 2 * DIM), lambda i: (i, 0)),
    )(wt, wlast)
    return paired.reshape(V_ROWS, DIM)


@jax.jit
def kernel(weight, indices, offsets):
    indices = indices.astype(jnp.int32)
    offsets = offsets.astype(jnp.int32)
    weight_rm = _relayout(weight)
    indices_p = jnp.concatenate(
        [indices, jnp.zeros((IDX_PAD - N_IDX,), jnp.int32)])
    offsets_p = jnp.concatenate(
        [offsets, jnp.full((OFF_PAD - (BATCH + 1),), N_IDX, jnp.int32)])

    mesh = plsc.VectorSubcoreMesh(core_axis_name="c", subcore_axis_name="s")
    out = pl.kernel(
        _sc_body,
        out_type=jax.ShapeDtypeStruct((BATCH, DIM), jnp.float32),
        mesh=mesh,
        compiler_params=pltpu.CompilerParams(use_tc_tiling_on_sc=False),
        scratch_types=[
            pltpu.VMEM((SEG_P,), jnp.int32),
            pltpu.VMEM((SEG_ROWS, 128), jnp.int32),
            pltpu.VMEM((OFFV,), jnp.int32),
            pltpu.VMEM((SEG_P // 16,), jnp.int32),
            pltpu.VMEM((SEG_P // 16,), jnp.int32),
            pltpu.VMEM((2, 128, DIM), jnp.float32),
            pltpu.VMEM_SHARED((NS * BLK, DIM), jnp.float32),
            pltpu.SemaphoreType.DMA,
            pltpu.SemaphoreType.DMA,
        ],
    )(weight_rm, indices_p, offsets_p)
    return out


# speedup vs baseline: 2.3734x; 1.2650x over previous
import jax
import jax.numpy as jnp
from jax import lax
from jax.experimental import pallas as pl
from jax.experimental.pallas import tpu as pltpu
from jax.experimental.pallas import tpu_sc as plsc

NUM_EMB = 1000000
DIM = 64
BATCH = 16384
N_IDX = 327680

NC = 2
NS = 16
NW = NC * NS
BAGS_W = BATCH // NW
BLK = BAGS_W + 8
DUMP = BAGS_W
SEG_P = 16384
SEG_ROWS = SEG_P // 128
NBV = SEG_P // 16 // 16
OFFV = 528
OFF_PAD = 31 * BAGS_W + OFFV
IDX_PAD = N_IDX + SEG_P


def _sc_body(weight, indices, offsets, out, idx_v, seg_v, off_v, bv_v, pv_v,
             rows_v, acc, sem0, sem1):
    c = lax.axis_index("c")
    s = lax.axis_index("s")
    wid = c * NS + s
    blk0 = s * BLK

    iota16 = lax.iota(jnp.int32, 16)
    zi16 = jnp.zeros((16,), jnp.int32)
    zf16 = jnp.zeros((16,), jnp.float32)
    ones16 = jnp.ones((16,), jnp.int32)

    def zrow(t, _):
        rows_v[0, t >> 2, pl.ds((t & 3) * 16, 16)] = zf16
        return 0

    lax.fori_loop(0, 128 * DIM // 16, zrow, 0)
    for k in range(4):
        pltpu.sync_copy(rows_v.at[0], acc.at[pl.ds(blk0 + k * 128, 128)])
    pltpu.sync_copy(rows_v.at[0, pl.ds(0, 8)], acc.at[pl.ds(blk0 + 512, 8)])

    pltpu.sync_copy(offsets.at[pl.ds(wid * BAGS_W, OFFV)], off_v)
    first = off_v[pl.ds(0, 16)][0]
    last = off_v[pl.ds(512, 16)][0]
    pstart = jnp.where(wid == 0, 0, first)
    pend = jnp.where(wid == NW - 1, N_IDX, last)
    a0 = (pstart >> 7) << 7
    nsc = jnp.maximum(0, (pend - a0 + SEG_P - 1) >> 14)

    def superchunk(sc_i, _):
        a0_sc = pl.multiple_of(a0 + sc_i * SEG_P, 128)

        def zseg(t, _):
            seg_v[t >> 3, pl.ds((t & 7) * 16, 16)] = zi16
            return 0

        lax.fori_loop(0, SEG_P // 16, zseg, 0)

        def zb(u, _):
            bv_v[pl.ds(u * 16, 16)] = zi16
            return 0

        lax.fori_loop(0, NBV, zb, 0)

        pltpu.sync_copy(indices.at[pl.ds(a0_sc, SEG_P)], idx_v)

        def remap(t, _):
            v = idx_v[pl.ds(t * 16, 16)]
            r = v & 4095
            h = jnp.where(r >= 2048, ones16, zi16)
            idx_v[pl.ds(t * 16, 16)] = (v - r) + ((r & 2047) << 1) + h
            return 0

        lax.fori_loop(0, SEG_P // 16, remap, 0)

        def do_boundary(k, v_k, base):
            v_k = jnp.where(
                jnp.logical_and(k == BAGS_W, wid == NW - 1), N_IDX, v_k)
            rel = v_k - a0_sc
            below = rel < 0
            inr = jnp.logical_and(rel >= 0, rel < SEG_P)
            relc = jnp.clip(rel, 0, SEG_P - 1)
            row = relc >> 7
            cg = (relc >> 4) & 7
            lane = relc & 15
            inr_i = jnp.where(inr, 1, 0)
            cur = seg_v[row, pl.ds(cg * 16, 16)]
            ramp = jnp.where(iota16 >= lane, ones16, zi16) * inr_i
            seg_v[row, pl.ds(cg * 16, 16)] = cur + ramp
            tv = relc >> 4
            boff = (tv >> 4) * 16
            bl = tv & 15
            bcur = bv_v[pl.ds(boff, 16)]
            binc = jnp.where(iota16 == bl, ones16, zi16) * inr_i
            bv_v[pl.ds(boff, 16)] = bcur + binc
            return base + jnp.where(below, 1, 0)

        def bgroup(kk, base):
            va = off_v[pl.ds(kk * 16, 16)]
            for u in range(16):
                k = kk * 16 + u
                is_valid = jnp.logical_and(k >= 1, k <= BAGS_W)
                v_k = jnp.where(is_valid, va[u], jnp.int32(2 * N_IDX))
                base = do_boundary(k, v_k, base)
            return base

        base = lax.fori_loop(0, (BAGS_W + 16) // 16, bgroup, jnp.int32(0))

        def prefix(u, carry):
            bv = bv_v[pl.ds(u * 16, 16)]
            incl = bv
            for sh in (1, 2, 4, 8):
                shifted = jnp.take(incl, jnp.clip(iota16 - sh, 0, 15))
                incl = incl + jnp.where(iota16 >= sh, shifted, zi16)
            pv_v[pl.ds(u * 16, 16)] = (incl - bv) + carry
            return carry + incl[15]

        lax.fori_loop(0, NBV, prefix, base)

        def finalize(g, _):
            pv = pv_v[pl.ds(g * 16, 16)]
            for u in range(16):
                t = g * 16 + u
                row = t >> 3
                cg = t & 7
                val = seg_v[row, pl.ds(cg * 16, 16)] + pv[u]
                p16 = (a0_sc + t * 16) + iota16
                val = jnp.where(p16 < pstart,
                                jnp.full((16,), DUMP, jnp.int32), val)
                seg_v[row, pl.ds(cg * 16, 16)] = val + blk0
            return 0

        lax.fori_loop(0, NBV, finalize, 0)

        nch = jnp.clip((pend - a0_sc + 127) >> 7, 0, SEG_ROWS)
        nch2 = nch >> 1

        def g_issue(j, b_ref, sem):
            pltpu.async_copy(
                weight.at[idx_v.at[pl.ds(j * 128, 128)]], b_ref, sem)

        def g_wait(b_ref, sem):
            pltpu.make_async_copy(
                weight.at[idx_v.at[pl.ds(0, 128)]], b_ref, sem).wait()

        g_issue(0, rows_v.at[0], sem0)

        def pair(m, _):
            j0 = m * 2
            g_issue(j0 + 1, rows_v.at[1], sem1)
            g_wait(rows_v.at[0], sem0)
            pltpu.sync_copy(rows_v.at[0], acc.at[seg_v.at[j0]], add=True)
            g_issue(jnp.minimum(j0 + 2, nch - 1), rows_v.at[0], sem0)
            g_wait(rows_v.at[1], sem1)
            pltpu.sync_copy(rows_v.at[1], acc.at[seg_v.at[j0 + 1]], add=True)
            return 0

        lax.fori_loop(0, nch2, pair, 0)

        def odd(_t, _c):
            g_wait(rows_v.at[0], sem0)
            pltpu.sync_copy(rows_v.at[0], acc.at[seg_v.at[nch - 1]], add=True)
            return 0

        lax.fori_loop(0, nch & 1, odd, 0)

        def drain(_t, _c):
            g_wait(rows_v.at[0], sem0)
            return 0

        lax.fori_loop(0, ((nch & 1) ^ 1) * jnp.minimum(nch2, 1), drain, 0)
        return 0

    lax.fori_loop(0, nsc, superchunk, 0)

    pltpu.sync_copy(acc.at[pl.ds(blk0, BAGS_W)],
                    out.at[pl.ds(wid * BAGS_W, BAGS_W)])


TBLK = 2048
GROUPS = (NUM_EMB + 2 * TBLK - 1) // (2 * TBLK)


STEP_COLS = 8 * TBLK
NFULL = NUM_EMB // STEP_COLS
NSTEP = GROUPS // 4 + 1
V_ROWS = NSTEP * STEP_COLS


def _tc_transpose_body(a_ref, last_ref, o_ref):
    i = pl.program_id(0)
    src = jnp.where(i == NSTEP - 1, last_ref[...], a_ref[...])
    for j in range(2):
        o_ref[pl.ds(j * TBLK, TBLK), 0:DIM] = (
            src[:, j * 2 * TBLK:j * 2 * TBLK + TBLK].T)
        o_ref[pl.ds(j * TBLK, TBLK), DIM:2 * DIM] = (
            src[:, j * 2 * TBLK + TBLK:(j + 1) * 2 * TBLK].T)


def _relayout(weight):
    wt = weight.T
    tail = jax.lax.slice(wt, (0, NFULL * STEP_COLS), (DIM, NUM_EMB))
    wlast = jnp.concatenate(
        [tail,
         jnp.zeros((DIM, STEP_COLS - (NUM_EMB - NFULL * STEP_COLS)),
                   jnp.float32)], axis=1)
    paired = pl.pallas_call(
        _tc_transpose_body,
        out_shape=jax.ShapeDtypeStruct((NSTEP * 4 * TBLK, 2 * DIM),
                                       jnp.float32),
        grid=(NSTEP,),
        in_specs=[
            pl.BlockSpec((DIM, STEP_COLS),
                         lambda i: (0, jnp.minimum(i, NFULL - 1))),
            pl.BlockSpec((DIM, STEP_COLS), lambda i: (0, 0)),
        ],
        out_specs=pl.BlockSpec((4 * TBLK, 2 * DIM), lambda i: (i, 0)),
    )(wt, wlast)
    return paired.reshape(V_ROWS, DIM)


@jax.jit
def kernel(weight, indices, offsets):
    indices = indices.astype(jnp.int32)
    offsets = offsets.astype(jnp.int32)
    weight_rm = _relayout(weight)
    indices_p = jnp.concatenate(
        [indices, jnp.zeros((IDX_PAD - N_IDX,), jnp.int32)])
    offsets_p = jnp.concatenate(
        [offsets, jnp.full((OFF_PAD - (BATCH + 1),), N_IDX, jnp.int32)])

    mesh = plsc.VectorSubcoreMesh(core_axis_name="c", subcore_axis_name="s")
    out = pl.kernel(
        _sc_body,
        out_type=jax.ShapeDtypeStruct((BATCH, DIM), jnp.float32),
        mesh=mesh,
        compiler_params=pltpu.CompilerParams(use_tc_tiling_on_sc=False),
        scratch_types=[
            pltpu.VMEM((SEG_P,), jnp.int32),
            pltpu.VMEM((SEG_ROWS, 128), jnp.int32),
            pltpu.VMEM((OFFV,), jnp.int32),
            pltpu.VMEM((SEG_P // 16,), jnp.int32),
            pltpu.VMEM((SEG_P // 16,), jnp.int32),
            pltpu.VMEM((2, 128, DIM), jnp.float32),
            pltpu.VMEM_SHARED((NS * BLK, DIM), jnp.float32),
            pltpu.SemaphoreType.DMA,
            pltpu.SemaphoreType.DMA,
        ],
    )(weight_rm, indices_p, offsets_p)
    return out
